# baseline (speedup 1.0000x reference)
import functools

import jax
import jax.numpy as jnp
from jax import lax
from jax.experimental import pallas as pl
from jax.experimental.pallas import tpu as pltpu
from jax.experimental.pallas import tpu_sc as plsc

_N = 10000
_E = 320000
_D = 128
_H = 128
_C = 64

_NC = 2
_NS = 16
_NW = _NC * _NS
_EPW = _E // _NW
_B = 80
_NITER = _EPW // _B
_RPS = 632
_NPAD = _RPS * _NS


_G = 4
_NGRP = 31


def _sc_agg_body(x_hbm, src_hbm, dst_hbm, zeros_hbm, out_hbm,
                 sA0, sA1, sA2, sA3, dA0, dA1, dA2, dA3,
                 sB0, sB1, sB2, sB3, dB0, dB1, dB2, dB3,
                 sC0, sC1, sC2, sC3, dC0, dC1, dC2, dC3,
                 rows0, rows1, rows2, rows3, agg_sh,
                 semIA, semIB, semIC,
                 sg0, sg1, sg2, sg3, ss0, ss1, ss2, ss3):
    c = lax.axis_index("c")
    s = lax.axis_index("s")
    w = c * _NS + s
    sets = (((sA0, sA1, sA2, sA3), (dA0, dA1, dA2, dA3), semIA),
            ((sB0, sB1, sB2, sB3), (dB0, dB1, dB2, dB3), semIB),
            ((sC0, sC1, sC2, sC3), (dC0, dC1, dC2, dC3), semIC))
    rows = (rows0, rows1, rows2, rows3)
    sg = (sg0, sg1, sg2, sg3)
    ss = (ss0, ss1, ss2, ss3)

    def load_group(g, si):
        srcs, dsts, semi = sets[si]
        base = w * _EPW + g * (_G * _B)
        for t in range(_G):
            pltpu.async_copy(src_hbm.at[pl.ds(base + t * _B, _B)], srcs[t],
                             semi)
            pltpu.async_copy(dst_hbm.at[pl.ds(base + t * _B, _B)], dsts[t],
                             semi)

    def wait_group_idx(si):
        srcs, dsts, semi = sets[si]
        for t in range(_G):
            pltpu.make_async_copy(src_hbm.at[pl.ds(0, _B)], srcs[t],
                                  semi).wait()
            pltpu.make_async_copy(src_hbm.at[pl.ds(0, _B)], dsts[t],
                                  semi).wait()

    def drain_scatter(t, si):
        pltpu.make_async_copy(rows[t], agg_sh.at[sets[si][1][t]],
                              ss[t]).wait()

    def process(si, prev_si):
        srcs, dsts, _ = sets[si]
        wait_group_idx(si)
        gd = [pltpu.async_copy(x_hbm.at[srcs[t]], rows[t], sg[t])
              for t in range(_G)]
        sd = []
        for t in range(_G):
            gd[t].wait()
            sd.append(pltpu.async_copy(rows[t], agg_sh.at[dsts[t]], ss[t],
                                       add=True))
        for d in sd:
            d.wait()

    load_group(0, 0)
    load_group(1, 1)
    load_group(2, 2)
    pltpu.sync_copy(zeros_hbm.at[pl.ds(s * _RPS, _RPS)],
                    agg_sh.at[pl.ds(s * _RPS, _RPS)])
    plsc.subcore_barrier()

    process(0, None)
    load_group(3, 0)
    process(1, 0)

    def body(k, carry):
        process(2, 1)
        load_group(3 * k + 4, 1)
        process(0, 2)
        load_group(3 * k + 5, 2)
        process(1, 0)
        load_group(3 * k + 6, 0)
        return carry

    lax.fori_loop(0, 9, body, 0)

    process(2, 1)
    process(0, 2)

    base = w * _EPW + (_NITER - 1) * _B
    pltpu.sync_copy(src_hbm.at[pl.ds(base, _B)], sA0)
    pltpu.sync_copy(dst_hbm.at[pl.ds(base, _B)], dA0)
    pltpu.async_copy(x_hbm.at[sA0], rows0, sg0).wait()
    pltpu.sync_copy(rows0, agg_sh.at[dA0], add=True)

    plsc.subcore_barrier()
    pltpu.sync_copy(agg_sh.at[pl.ds(s * _RPS, _RPS)],
                    out_hbm.at[c, pl.ds(s * _RPS, _RPS)])


@jax.jit
def _sc_segment_sum(x, src, dst, zeros):
    mesh = plsc.VectorSubcoreMesh(core_axis_name="c", subcore_axis_name="s")
    f = pl.kernel(
        _sc_agg_body,
        out_type=jax.ShapeDtypeStruct((_NC, _NPAD, _D), jnp.float32),
        mesh=mesh,
        scratch_types=[pltpu.VMEM((_B,), jnp.int32)] * 24
        + [pltpu.VMEM((_B, _D), jnp.float32)] * 4
        + [pltpu.VMEM_SHARED((_NPAD, _D), jnp.float32)]
        + [pltpu.SemaphoreType.DMA] * 11,
    )
    return f(x, src, dst, zeros)


def _mlp_body(eps_ref, x_ref, agg_ref, wa_ref, ba_ref, g_ref, be_ref,
              wb_ref, bb_ref, o_ref, *, last):
    agg = agg_ref[0, :_N, :] + agg_ref[1, :_N, :]
    h = (1.0 + eps_ref[0]) * x_ref[...] + agg
    t = jnp.dot(h, wa_ref[...], preferred_element_type=jnp.float32) + ba_ref[...]
    t = jnp.maximum(t, 0.0)
    mu = jnp.mean(t, axis=0, keepdims=True)
    var = jnp.mean((t - mu) ** 2, axis=0, keepdims=True)
    t = g_ref[...] * (t - mu) * lax.rsqrt(var + 1e-5) + be_ref[...]
    o = jnp.dot(t, wb_ref[...], preferred_element_type=jnp.float32) + bb_ref[...]
    if last:
        o = o - jnp.max(o, axis=-1, keepdims=True)
        o = o - jnp.log(jnp.sum(jnp.exp(o), axis=-1, keepdims=True))
    else:
        o = jnp.maximum(o, 0.0)
    o_ref[...] = o


def _mlp(eps, x, agg, wa, ba, g, be, wb, bb, *, last):
    cout = wb.shape[1]
    return pl.pallas_call(
        functools.partial(_mlp_body, last=last),
        out_shape=jax.ShapeDtypeStruct((_N, cout), jnp.float32),
        in_specs=[pl.BlockSpec(memory_space=pltpu.SMEM)]
        + [pl.BlockSpec(memory_space=pltpu.VMEM)] * 8,
        out_specs=pl.BlockSpec(memory_space=pltpu.VMEM),
    )(eps, x, agg, wa, ba, g, be, wb, bb)


def kernel(x, edge_index, eps1, W1a, b1a, g1, be1, W1b, b1b,
           eps2, W2a, b2a, g2, be2, W2b, b2b):
    ei = edge_index.astype(jnp.int32)
    zeros = jnp.zeros((_NPAD, _D), jnp.float32)
    e1 = jnp.reshape(eps1, (1,)).astype(jnp.float32)
    e2 = jnp.reshape(eps2, (1,)).astype(jnp.float32)

    src, dst = ei[0], ei[1]
    agg1 = _sc_segment_sum(x, src, dst, zeros)
    h1 = _mlp(e1, x, agg1, W1a, b1a.reshape(1, _H), g1.reshape(1, _H),
              be1.reshape(1, _H), W1b, b1b.reshape(1, _H), last=False)
    agg2 = _sc_segment_sum(h1, src, dst, zeros)
    out = _mlp(e2, h1, agg2, W2a, b2a.reshape(1, _H), g2.reshape(1, _H),
               be2.reshape(1, _H), W2b, b2b.reshape(1, _C), last=True)
    return out


# speedup vs baseline: 1.1339x; 1.1339x over previous
import functools

import jax
import jax.numpy as jnp
from jax import lax
from jax.experimental import pallas as pl
from jax.experimental.pallas import tpu as pltpu
from jax.experimental.pallas import tpu_sc as plsc

_N = 10000
_E = 320000
_D = 128
_H = 128
_C = 64

_NC = 2
_NS = 16
_NW = _NC * _NS
_EPW = _E // _NW
_B = 80
_NITER = _EPW // _B
_RPS = 632
_NPAD = _RPS * _NS


_G = 4
_NGRP = 31


def _sc_agg_body(x_hbm, src_hbm, dst_hbm, zeros_hbm, out_hbm,
                 sA0, sA1, sA2, sA3, dA0, dA1, dA2, dA3,
                 sB0, sB1, sB2, sB3, dB0, dB1, dB2, dB3,
                 sC0, sC1, sC2, sC3, dC0, dC1, dC2, dC3,
                 rows0, rows1, rows2, rows3, agg_sh,
                 semIA, semIB, semIC,
                 sg0, sg1, sg2, sg3, ss0, ss1, ss2, ss3):
    c = lax.axis_index("c")
    s = lax.axis_index("s")
    w = c * _NS + s
    sets = (((sA0, sA1, sA2, sA3), (dA0, dA1, dA2, dA3), semIA),
            ((sB0, sB1, sB2, sB3), (dB0, dB1, dB2, dB3), semIB),
            ((sC0, sC1, sC2, sC3), (dC0, dC1, dC2, dC3), semIC))
    rows = (rows0, rows1, rows2, rows3)
    sg = (sg0, sg1, sg2, sg3)
    ss = (ss0, ss1, ss2, ss3)

    def load_group(g, si):
        srcs, dsts, semi = sets[si]
        base = w * _EPW + g * (_G * _B)
        for t in range(_G):
            pltpu.async_copy(src_hbm.at[pl.ds(base + t * _B, _B)], srcs[t],
                             semi)
            pltpu.async_copy(dst_hbm.at[pl.ds(base + t * _B, _B)], dsts[t],
                             semi)

    def wait_group_idx(si):
        srcs, dsts, semi = sets[si]
        for t in range(_G):
            pltpu.make_async_copy(src_hbm.at[pl.ds(0, _B)], srcs[t],
                                  semi).wait()
            pltpu.make_async_copy(src_hbm.at[pl.ds(0, _B)], dsts[t],
                                  semi).wait()

    def drain_scatter(t, si):
        pltpu.make_async_copy(rows[t], agg_sh.at[sets[si][1][t]],
                              ss[t]).wait()

    def process(si, sd_prev=None):
        srcs, dsts, _ = sets[si]
        wait_group_idx(si)
        gd = []
        for t in range(_G):
            if sd_prev is not None:
                sd_prev[t].wait()
            gd.append(pltpu.async_copy(x_hbm.at[srcs[t]], rows[t], sg[t]))
        sd = []
        for t in range(_G):
            gd[t].wait()
            sd.append(pltpu.async_copy(rows[t], agg_sh.at[dsts[t]], ss[t],
                                       add=True))
        return sd

    def drain(sd):
        for d in sd:
            d.wait()

    load_group(0, 0)
    load_group(1, 1)
    load_group(2, 2)
    pltpu.sync_copy(zeros_hbm.at[pl.ds(s * _RPS, _RPS)],
                    agg_sh.at[pl.ds(s * _RPS, _RPS)])
    plsc.subcore_barrier()

    drain(process(0))
    load_group(3, 0)
    drain(process(1))

    def body(k, carry):
        sd_c = process(2)
        load_group(3 * k + 4, 1)
        sd_a = process(0, sd_c)
        load_group(3 * k + 5, 2)
        sd_b = process(1, sd_a)
        load_group(3 * k + 6, 0)
        drain(sd_b)
        return carry

    lax.fori_loop(0, 9, body, 0)

    sd_c = process(2)
    sd_a = process(0, sd_c)
    drain(sd_a)

    base = w * _EPW + (_NITER - 1) * _B
    pltpu.sync_copy(src_hbm.at[pl.ds(base, _B)], sA0)
    pltpu.sync_copy(dst_hbm.at[pl.ds(base, _B)], dA0)
    pltpu.async_copy(x_hbm.at[sA0], rows0, sg0).wait()
    pltpu.sync_copy(rows0, agg_sh.at[dA0], add=True)

    plsc.subcore_barrier()
    pltpu.sync_copy(agg_sh.at[pl.ds(s * _RPS, _RPS)],
                    out_hbm.at[c, pl.ds(s * _RPS, _RPS)])


@jax.jit
def _sc_segment_sum(x, src, dst, zeros):
    mesh = plsc.VectorSubcoreMesh(core_axis_name="c", subcore_axis_name="s")
    f = pl.kernel(
        _sc_agg_body,
        out_type=jax.ShapeDtypeStruct((_NC, _NPAD, _D), jnp.float32),
        mesh=mesh,
        scratch_types=[pltpu.VMEM((_B,), jnp.int32)] * 24
        + [pltpu.VMEM((_B, _D), jnp.float32)] * 4
        + [pltpu.VMEM_SHARED((_NPAD, _D), jnp.float32)]
        + [pltpu.SemaphoreType.DMA] * 11,
    )
    return f(x, src, dst, zeros)


def _mlp_body(eps_ref, x_ref, agg_ref, wa_ref, ba_ref, g_ref, be_ref,
              wb_ref, bb_ref, o_ref, *, last):
    agg = agg_ref[0, :_N, :] + agg_ref[1, :_N, :]
    h = (1.0 + eps_ref[0]) * x_ref[...] + agg
    t = jnp.dot(h, wa_ref[...], preferred_element_type=jnp.float32) + ba_ref[...]
    t = jnp.maximum(t, 0.0)
    mu = jnp.mean(t, axis=0, keepdims=True)
    var = jnp.mean((t - mu) ** 2, axis=0, keepdims=True)
    t = g_ref[...] * (t - mu) * lax.rsqrt(var + 1e-5) + be_ref[...]
    o = jnp.dot(t, wb_ref[...], preferred_element_type=jnp.float32) + bb_ref[...]
    if last:
        o = o - jnp.max(o, axis=-1, keepdims=True)
        o = o - jnp.log(jnp.sum(jnp.exp(o), axis=-1, keepdims=True))
    else:
        o = jnp.maximum(o, 0.0)
    o_ref[...] = o


def _mlp(eps, x, agg, wa, ba, g, be, wb, bb, *, last):
    cout = wb.shape[1]
    return pl.pallas_call(
        functools.partial(_mlp_body, last=last),
        out_shape=jax.ShapeDtypeStruct((_N, cout), jnp.float32),
        in_specs=[pl.BlockSpec(memory_space=pltpu.SMEM)]
        + [pl.BlockSpec(memory_space=pltpu.VMEM)] * 8,
        out_specs=pl.BlockSpec(memory_space=pltpu.VMEM),
    )(eps, x, agg, wa, ba, g, be, wb, bb)


def kernel(x, edge_index, eps1, W1a, b1a, g1, be1, W1b, b1b,
           eps2, W2a, b2a, g2, be2, W2b, b2b):
    ei = edge_index.astype(jnp.int32)
    zeros = jnp.zeros((_NPAD, _D), jnp.float32)
    e1 = jnp.reshape(eps1, (1,)).astype(jnp.float32)
    e2 = jnp.reshape(eps2, (1,)).astype(jnp.float32)

    src, dst = ei[0], ei[1]
    agg1 = _sc_segment_sum(x, src, dst, zeros)
    h1 = _mlp(e1, x, agg1, W1a, b1a.reshape(1, _H), g1.reshape(1, _H),
              be1.reshape(1, _H), W1b, b1b.reshape(1, _H), last=False)
    agg2 = _sc_segment_sum(h1, src, dst, zeros)
    out = _mlp(e2, h1, agg2, W2a, b2a.reshape(1, _H), g2.reshape(1, _H),
               be2.reshape(1, _H), W2b, b2b.reshape(1, _C), last=True)
    return out


# speedup vs baseline: 1.1968x; 1.0555x over previous
import functools

import jax
import jax.numpy as jnp
from jax import lax
from jax.experimental import pallas as pl
from jax.experimental.pallas import tpu as pltpu
from jax.experimental.pallas import tpu_sc as plsc

_N = 10000
_E = 320000
_D = 128
_H = 128
_C = 64

_NC = 2
_NS = 16
_NW = _NC * _NS
_EPW = _E // _NW
_B = 80
_NITER = _EPW // _B
_RPS = 632
_NPAD = _RPS * _NS


_G = 4
_NGRP = 31


def _sc_agg_body(x_hbm, src_hbm, dst_hbm, zeros_hbm, out_hbm,
                 sA0, sA1, sA2, sA3, dA0, dA1, dA2, dA3,
                 sB0, sB1, sB2, sB3, dB0, dB1, dB2, dB3,
                 sC0, sC1, sC2, sC3, dC0, dC1, dC2, dC3,
                 rows0, rows1, rows2, rows3, agg_sh,
                 semIA, semIB, semIC,
                 sg0, sg1, sg2, sg3, ss0, ss1, ss2, ss3):
    c = lax.axis_index("c")
    s = lax.axis_index("s")
    w = c * _NS + s
    sets = (((sA0, sA1, sA2, sA3), (dA0, dA1, dA2, dA3), semIA),
            ((sB0, sB1, sB2, sB3), (dB0, dB1, dB2, dB3), semIB),
            ((sC0, sC1, sC2, sC3), (dC0, dC1, dC2, dC3), semIC))
    rows = (rows0, rows1, rows2, rows3)
    sg = (sg0, sg1, sg2, sg3)
    ss = (ss0, ss1, ss2, ss3)

    def load_group(g, si):
        srcs, dsts, semi = sets[si]
        base = w * _EPW + g * (_G * _B)
        for t in range(_G):
            pltpu.async_copy(src_hbm.at[pl.ds(base + t * _B, _B)], srcs[t],
                             semi)
            pltpu.async_copy(dst_hbm.at[pl.ds(base + t * _B, _B)], dsts[t],
                             semi)

    def wait_group_idx(si):
        srcs, dsts, semi = sets[si]
        for t in range(_G):
            pltpu.make_async_copy(src_hbm.at[pl.ds(0, _B)], srcs[t],
                                  semi).wait()
            pltpu.make_async_copy(src_hbm.at[pl.ds(0, _B)], dsts[t],
                                  semi).wait()

    def drain_scatter(t, si):
        pltpu.make_async_copy(rows[t], agg_sh.at[sets[si][1][t]],
                              ss[t]).wait()

    def process(si, sd_prev=None):
        srcs, dsts, _ = sets[si]
        wait_group_idx(si)
        gd = []
        for t in range(_G):
            if sd_prev is not None:
                sd_prev[t].wait()
            gd.append(pltpu.async_copy(x_hbm.at[srcs[t]], rows[t], sg[t]))
        sd = []
        for t in range(_G):
            gd[t].wait()
            sd.append(pltpu.async_copy(rows[t], agg_sh.at[dsts[t]], ss[t],
                                       add=True))
        return sd

    def drain(sd):
        for d in sd:
            d.wait()

    load_group(0, 0)
    load_group(1, 1)
    load_group(2, 2)
    pltpu.sync_copy(zeros_hbm.at[pl.ds(s * _RPS, _RPS)],
                    agg_sh.at[pl.ds(s * _RPS, _RPS)])
    plsc.subcore_barrier()

    sd = None
    for g in range(_NGRP):
        sd = process(g % 3, sd)
        if 1 <= g <= _NGRP - 3:
            load_group(g + 2, (g + 2) % 3)
    drain(sd)

    base = w * _EPW + (_NITER - 1) * _B
    pltpu.sync_copy(src_hbm.at[pl.ds(base, _B)], sA0)
    pltpu.sync_copy(dst_hbm.at[pl.ds(base, _B)], dA0)
    pltpu.async_copy(x_hbm.at[sA0], rows0, sg0).wait()
    pltpu.sync_copy(rows0, agg_sh.at[dA0], add=True)

    plsc.subcore_barrier()
    pltpu.sync_copy(agg_sh.at[pl.ds(s * _RPS, _RPS)],
                    out_hbm.at[c, pl.ds(s * _RPS, _RPS)])


@jax.jit
def _sc_segment_sum(x, src, dst, zeros):
    mesh = plsc.VectorSubcoreMesh(core_axis_name="c", subcore_axis_name="s")
    f = pl.kernel(
        _sc_agg_body,
        out_type=jax.ShapeDtypeStruct((_NC, _NPAD, _D), jnp.float32),
        mesh=mesh,
        scratch_types=[pltpu.VMEM((_B,), jnp.int32)] * 24
        + [pltpu.VMEM((_B, _D), jnp.float32)] * 4
        + [pltpu.VMEM_SHARED((_NPAD, _D), jnp.float32)]
        + [pltpu.SemaphoreType.DMA] * 11,
    )
    return f(x, src, dst, zeros)


def _mlp_body(eps_ref, x_ref, agg_ref, wa_ref, ba_ref, g_ref, be_ref,
              wb_ref, bb_ref, o_ref, *, last):
    agg = agg_ref[0, :_N, :] + agg_ref[1, :_N, :]
    h = (1.0 + eps_ref[0]) * x_ref[...] + agg
    t = jnp.dot(h, wa_ref[...], preferred_element_type=jnp.float32) + ba_ref[...]
    t = jnp.maximum(t, 0.0)
    mu = jnp.mean(t, axis=0, keepdims=True)
    var = jnp.mean((t - mu) ** 2, axis=0, keepdims=True)
    t = g_ref[...] * (t - mu) * lax.rsqrt(var + 1e-5) + be_ref[...]
    o = jnp.dot(t, wb_ref[...], preferred_element_type=jnp.float32) + bb_ref[...]
    if last:
        o = o - jnp.max(o, axis=-1, keepdims=True)
        o = o - jnp.log(jnp.sum(jnp.exp(o), axis=-1, keepdims=True))
    else:
        o = jnp.maximum(o, 0.0)
    o_ref[...] = o


def _mlp(eps, x, agg, wa, ba, g, be, wb, bb, *, last):
    cout = wb.shape[1]
    return pl.pallas_call(
        functools.partial(_mlp_body, last=last),
        out_shape=jax.ShapeDtypeStruct((_N, cout), jnp.float32),
        in_specs=[pl.BlockSpec(memory_space=pltpu.SMEM)]
        + [pl.BlockSpec(memory_space=pltpu.VMEM)] * 8,
        out_specs=pl.BlockSpec(memory_space=pltpu.VMEM),
    )(eps, x, agg, wa, ba, g, be, wb, bb)


def kernel(x, edge_index, eps1, W1a, b1a, g1, be1, W1b, b1b,
           eps2, W2a, b2a, g2, be2, W2b, b2b):
    ei = edge_index.astype(jnp.int32)
    zeros = jnp.zeros((_NPAD, _D), jnp.float32)
    e1 = jnp.reshape(eps1, (1,)).astype(jnp.float32)
    e2 = jnp.reshape(eps2, (1,)).astype(jnp.float32)

    src, dst = ei[0], ei[1]
    agg1 = _sc_segment_sum(x, src, dst, zeros)
    h1 = _mlp(e1, x, agg1, W1a, b1a.reshape(1, _H), g1.reshape(1, _H),
              be1.reshape(1, _H), W1b, b1b.reshape(1, _H), last=False)
    agg2 = _sc_segment_sum(h1, src, dst, zeros)
    out = _mlp(e2, h1, agg2, W2a, b2a.reshape(1, _H), g2.reshape(1, _H),
               be2.reshape(1, _H), W2b, b2b.reshape(1, _C), last=True)
    return out
